# Initial kernel scaffold; baseline (speedup 1.0000x reference)
import jax
import jax.numpy as jnp
from jax.experimental import pallas as pl


def kernel(embeddings, sp_seg, rot_sp, edges, weights, val_sp, rotation_angle):
    raise NotImplementedError("write your pallas kernel here")


# speedup vs baseline: 15.4570x; 15.4570x over previous
import functools

import jax
import jax.numpy as jnp
from jax import lax
from jax.experimental import pallas as pl
from jax.experimental.pallas import tpu as pltpu

_DELTA_VAR = 0.1
_DELTA_DIST = 0.3
_B = 2
_C = 128
_D = 32
_H = 64
_W = 64
_E = 512
_P = _H * _W


def _loss_kernel(emb_ref, seg_ref, edges_ref, w_ref, val_ref, ang_ref, out_ref):
    f32 = jnp.float32
    ids_p = lax.broadcasted_iota(jnp.int32, (_C, _P), 0)
    ids_sq = lax.broadcasted_iota(jnp.int32, (_C, _C), 1)
    ids_e = lax.broadcasted_iota(jnp.int32, (_E, _C), 1)
    ang = ang_ref[0, 0]
    loss = f32(0.0)
    for i in range(_B):
        val = val_ref[i]
        vm = (val[:, None] == ids_sq).astype(f32)
        mult = vm.sum(axis=0)[:, None]
        mcs = []
        intra_i = f32(0.0)
        for v in range(2):
            j = v * _B + i
            e = emb_ref[j]
            seg = seg_ref[j]
            mask = (ids_p == seg[None, :]).astype(f32)
            counts = mask.sum(axis=1)[:, None]
            sums = lax.dot_general(mask, e, (((1,), (1,)), ((), ())),
                                   preferred_element_type=f32)
            mean = sums / counts
            n0 = jnp.sqrt((mean[:, :16] ** 2).sum(axis=1)[:, None])
            n1 = jnp.sqrt((mean[:, 16:] ** 2).sum(axis=1)[:, None])
            mn = jnp.concatenate([mean[:, :16] / n0 + 1e-10,
                                  mean[:, 16:] / n1 + 1e-10], axis=1)
            dotmat = lax.dot_general(mn, e, (((1,), (0,)), ((), ())),
                                     preferred_element_type=f32)
            wquot = mult / counts
            intra_i = intra_i + (mask
                                 * jnp.maximum((2.0 - dotmat) * 0.5 - _DELTA_VAR, 0.0)
                                 * wquot).sum() / _C
            mc = lax.dot_general(vm, mn, (((1,), (0,)), ((), ())),
                                 preferred_element_type=f32)
            mcs.append(mc)
            e0 = edges_ref[2 * i + 0]
            e1 = edges_ref[2 * i + 1]
            oh0 = (e0[:, None] == ids_e).astype(f32)
            oh1 = (e1[:, None] == ids_e).astype(f32)
            a = lax.dot_general(oh0, mc[:, :16], (((1,), (0,)), ((), ())),
                                preferred_element_type=f32)
            b = lax.dot_general(oh1, mc[:, :16], (((1,), (0,)), ((), ())),
                                preferred_element_type=f32)
            inter = (1.0 - (a * b).sum(axis=1)) * w_ref[i]
            loss = loss + jnp.maximum(_DELTA_DIST - inter, 0.0).sum() / _E
        s0 = (mcs[0][:16] * mcs[1][:16]).sum(axis=0)
        rd1 = (1.0 - s0).mean()
        loss = loss + jnp.maximum(rd1 - _DELTA_VAR, 0.0)
        s1 = (mcs[0][16:] * mcs[1][16:]).sum(axis=0)
        rd2 = (1.0 - s1).mean()
        loss = loss + jnp.maximum(_DELTA_DIST - rd2, 0.0) * ang
        loss = loss + intra_i
    out_ref[0, 0] = loss


@functools.partial(jax.jit, static_argnames=("interpret",))
def _run(embeddings, sp_seg, rot_sp, edges, weights, val_sp, rotation_angle,
         interpret=False):
    emb = embeddings.reshape(2 * _B, _D, _P)
    seg_all = jnp.concatenate([sp_seg.reshape(_B, _P),
                               rot_sp.reshape(_B, _P)], axis=0)
    edges_all = edges.reshape(2 * _B, _E)
    ang = rotation_angle.reshape(1, 1)
    out = pl.pallas_call(
        _loss_kernel,
        out_shape=jax.ShapeDtypeStruct((1, 1), jnp.float32),
        in_specs=[
            pl.BlockSpec(memory_space=pltpu.VMEM),
            pl.BlockSpec(memory_space=pltpu.VMEM),
            pl.BlockSpec(memory_space=pltpu.VMEM),
            pl.BlockSpec(memory_space=pltpu.VMEM),
            pl.BlockSpec(memory_space=pltpu.VMEM),
            pl.BlockSpec(memory_space=pltpu.SMEM),
        ],
        out_specs=pl.BlockSpec(memory_space=pltpu.SMEM),
        interpret=interpret,
    )(emb, seg_all, edges_all, weights, val_sp, ang)
    return out[0, 0]


def kernel(embeddings, sp_seg, rot_sp, edges, weights, val_sp, rotation_angle):
    return _run(embeddings, sp_seg, rot_sp, edges, weights, val_sp,
                jnp.asarray(rotation_angle, jnp.float32))
